# baseline (speedup 1.0000x reference)
import functools

import jax
import jax.numpy as jnp
from jax import lax
from jax.experimental import pallas as pl
from jax.experimental.pallas import tpu as pltpu
from jax.experimental.pallas import tpu_sc as plsc

_N_ROWS = 100000
_D = 128
_NSEG = 512
_NC = 2
_NS = 16
_NW = _NC * _NS
_CHUNK = 128
_NFULL = _N_ROWS // _CHUNK
_TAIL = _N_ROWS - _NFULL * _CHUNK
_NJ = _NFULL // _NW
_NEXTRA = _NFULL - _NJ * _NW
_ZROWS = _NSEG // _NS
_NBUF = 6
_AHEAD = 3


def _sc_partial_segment_sum(feat, ids_flat):
    mesh = plsc.VectorSubcoreMesh(core_axis_name="c", subcore_axis_name="s")

    @functools.partial(
        pl.kernel,
        mesh=mesh,
        out_type=jax.ShapeDtypeStruct((_NC * _NSEG, _D), jnp.float32),
        scratch_types=(
            [pltpu.VMEM_SHARED((_NSEG, _D), jnp.float32)]
            + [pltpu.VMEM((_CHUNK, _D), jnp.float32)] * _NBUF
            + [pltpu.VMEM((_NJ + 1, _CHUNK), jnp.int32),
               pltpu.VMEM((_TAIL, _D), jnp.float32),
               pltpu.VMEM((_TAIL,), jnp.int32)]
            + [pltpu.SemaphoreType.DMA] * (2 * _NBUF + 1)
        ),
    )
    def k(feat_hbm, ids1_hbm, out_hbm, acc, *rest):
        bufs = rest[:_NBUF]
        idx_v, tbuf, tidx = rest[_NBUF:_NBUF + 3]
        lsems = rest[_NBUF + 3:2 * _NBUF + 3]
        ssems = rest[2 * _NBUF + 3:3 * _NBUF + 3]
        isem = rest[3 * _NBUF + 3]
        cid = lax.axis_index("c")
        sid = lax.axis_index("s")
        wid = cid * _NS + sid
        c0 = _NJ * wid + jnp.minimum(wid, _NEXTRA)

        id_loads = []
        for j in range(_NJ + 1):
            row = (c0 + j) * _CHUNK
            if j == _NJ:
                row = jnp.minimum(row, (_NFULL - 1) * _CHUNK)
            id_loads.append(pltpu.async_copy(
                ids1_hbm.at[pl.ds(pl.multiple_of(row, _CHUNK), _CHUNK)],
                idx_v.at[j], isem))

        zero = jnp.zeros((16,), jnp.float32)
        for r in range(_ZROWS):
            for l in range(_D // 16):
                bufs[0][r, pl.ds(l * 16, 16)] = zero
        pltpu.sync_copy(bufs[0].at[pl.ds(0, _ZROWS)],
                        acc.at[pl.ds(sid * _ZROWS, _ZROWS)])

        def start_load(j):
            row = pl.multiple_of((c0 + j) * _CHUNK, _CHUNK)
            return pltpu.async_copy(feat_hbm.at[pl.ds(row, _CHUNK)],
                                    bufs[j % _NBUF], lsems[j % _NBUF])

        loads = {j: start_load(j) for j in range(_AHEAD)}
        for h in id_loads:
            h.wait()
        plsc.subcore_barrier()
        scats = {}
        for j in range(_NJ):
            if j >= _AHEAD:
                scats.pop(j - _AHEAD).wait()
            if j + _AHEAD < _NJ:
                loads[j + _AHEAD] = start_load(j + _AHEAD)
            loads.pop(j).wait()
            scats[j] = pltpu.async_copy(bufs[j % _NBUF], acc.at[idx_v.at[j]],
                                        ssems[j % _NBUF], add=True)
        for j in sorted(scats):
            scats.pop(j).wait()

        @pl.when(wid < _NEXTRA)
        def _():
            start_load(_NJ).wait()
            pltpu.sync_copy(bufs[_NJ % _NBUF], acc.at[idx_v.at[_NJ]], add=True)

        @pl.when(wid == _NW - 1)
        def _():
            base = _NFULL * _CHUNK
            f = pltpu.async_copy(feat_hbm.at[pl.ds(base, _TAIL)], tbuf,
                                 lsems[0])
            i = pltpu.async_copy(ids1_hbm.at[pl.ds(base, _TAIL)], tidx,
                                 lsems[1])
            f.wait()
            i.wait()
            pltpu.sync_copy(tbuf, acc.at[tidx], add=True)

        plsc.subcore_barrier()
        pltpu.sync_copy(acc.at[pl.ds(sid * _ZROWS, _ZROWS)],
                        out_hbm.at[pl.ds(cid * _NSEG + sid * _ZROWS, _ZROWS)])

    return k(feat, ids_flat)


def _tc_merge(x_ref, o_ref):
    o_ref[...] = x_ref[0] + x_ref[1]


def kernel(feat, segment_ids, num_segments):
    ids = segment_ids.astype(jnp.int32)
    part = _sc_partial_segment_sum(feat, ids).reshape(_NC, _NSEG, _D)
    return pl.pallas_call(
        _tc_merge,
        out_shape=jax.ShapeDtypeStruct((_NSEG, _D), jnp.float32),
    )(part)


# speedup vs baseline: 1.1060x; 1.1060x over previous
import functools

import jax
import jax.numpy as jnp
from jax import lax
from jax.experimental import pallas as pl
from jax.experimental.pallas import tpu as pltpu
from jax.experimental.pallas import tpu_sc as plsc

_N_ROWS = 100000
_D = 128
_NSEG = 512
_NC = 2
_NS = 16
_NW = _NC * _NS
_CHUNK = 128
_NFULL = _N_ROWS // _CHUNK
_TAIL = _N_ROWS - _NFULL * _CHUNK
_NJ = _NFULL // _NW
_NEXTRA = _NFULL - _NJ * _NW
_ZROWS = _NSEG // _NS
_NBUF = 6
_AHEAD = 3


def _sc_partial_segment_sum(feat, ids_flat):
    mesh = plsc.VectorSubcoreMesh(core_axis_name="c", subcore_axis_name="s")

    @functools.partial(
        pl.kernel,
        mesh=mesh,
        out_type=jax.ShapeDtypeStruct((_NC * _NSEG, _D), jnp.float32),
        scratch_types=(
            [pltpu.VMEM_SHARED((_NSEG, _D), jnp.float32)]
            + [pltpu.VMEM((_CHUNK, _D), jnp.float32)] * _NBUF
            + [pltpu.VMEM((_NJ + 1, _CHUNK), jnp.int32),
               pltpu.VMEM((_TAIL, _D), jnp.float32),
               pltpu.VMEM((_TAIL,), jnp.int32)]
            + [pltpu.SemaphoreType.DMA] * (2 * _NBUF + 1)
        ),
    )
    def k(feat_hbm, ids1_hbm, out_hbm, acc, *rest):
        bufs = rest[:_NBUF]
        idx_v, tbuf, tidx = rest[_NBUF:_NBUF + 3]
        lsems = rest[_NBUF + 3:2 * _NBUF + 3]
        ssems = rest[2 * _NBUF + 3:3 * _NBUF + 3]
        isem = rest[3 * _NBUF + 3]
        cid = lax.axis_index("c")
        sid = lax.axis_index("s")
        wid = cid * _NS + sid

        id_loads = []
        for j in range(_NJ + 1):
            row = (j * _NW + wid) * _CHUNK
            if j == _NJ:
                row = jnp.minimum(row, (_NFULL - 1) * _CHUNK)
            id_loads.append(pltpu.async_copy(
                ids1_hbm.at[pl.ds(pl.multiple_of(row, _CHUNK), _CHUNK)],
                idx_v.at[j], isem))

        zero = jnp.zeros((16,), jnp.float32)
        for r in range(_ZROWS):
            for l in range(_D // 16):
                bufs[0][r, pl.ds(l * 16, 16)] = zero
        pltpu.sync_copy(bufs[0].at[pl.ds(0, _ZROWS)],
                        acc.at[pl.ds(sid * _ZROWS, _ZROWS)])

        def start_load(j):
            row = pl.multiple_of((j * _NW + wid) * _CHUNK, _CHUNK)
            return pltpu.async_copy(feat_hbm.at[pl.ds(row, _CHUNK)],
                                    bufs[j % _NBUF], lsems[j % _NBUF])

        loads = {j: start_load(j) for j in range(_AHEAD)}
        for h in id_loads:
            h.wait()
        plsc.subcore_barrier()
        scats = {}
        for j in range(_NJ):
            if j >= _AHEAD:
                scats.pop(j - _AHEAD).wait()
            if j + _AHEAD < _NJ:
                loads[j + _AHEAD] = start_load(j + _AHEAD)
            loads.pop(j).wait()
            scats[j] = pltpu.async_copy(bufs[j % _NBUF], acc.at[idx_v.at[j]],
                                        ssems[j % _NBUF], add=True)
        for j in sorted(scats):
            scats.pop(j).wait()

        @pl.when(wid < _NEXTRA)
        def _():
            start_load(_NJ).wait()
            pltpu.sync_copy(bufs[_NJ % _NBUF], acc.at[idx_v.at[_NJ]], add=True)

        @pl.when(wid == _NW - 1)
        def _():
            base = _NFULL * _CHUNK
            f = pltpu.async_copy(feat_hbm.at[pl.ds(base, _TAIL)], tbuf,
                                 lsems[0])
            i = pltpu.async_copy(ids1_hbm.at[pl.ds(base, _TAIL)], tidx,
                                 lsems[1])
            f.wait()
            i.wait()
            pltpu.sync_copy(tbuf, acc.at[tidx], add=True)

        plsc.subcore_barrier()
        pltpu.sync_copy(acc.at[pl.ds(sid * _ZROWS, _ZROWS)],
                        out_hbm.at[pl.ds(cid * _NSEG + sid * _ZROWS, _ZROWS)])

    return k(feat, ids_flat)


def _tc_merge(x_ref, o_ref):
    o_ref[...] = x_ref[0] + x_ref[1]


def kernel(feat, segment_ids, num_segments):
    ids = segment_ids.astype(jnp.int32)
    part = _sc_partial_segment_sum(feat, ids).reshape(_NC, _NSEG, _D)
    return pl.pallas_call(
        _tc_merge,
        out_shape=jax.ShapeDtypeStruct((_NSEG, _D), jnp.float32),
    )(part)


# speedup vs baseline: 1.1077x; 1.0015x over previous
import functools

import jax
import jax.numpy as jnp
from jax import lax
from jax.experimental import pallas as pl
from jax.experimental.pallas import tpu as pltpu
from jax.experimental.pallas import tpu_sc as plsc

_N_ROWS = 100000
_D = 128
_NSEG = 512
_NC = 2
_NS = 16
_NW = _NC * _NS
_CHUNK = 128
_NFULL = _N_ROWS // _CHUNK
_TAIL = _N_ROWS - _NFULL * _CHUNK
_NJ = _NFULL // _NW
_NEXTRA = _NFULL - _NJ * _NW
_ZROWS = _NSEG // _NS
_NBUF = 6
_AHEAD = 3


def _sc_partial_segment_sum(feat, ids_flat):
    mesh = plsc.VectorSubcoreMesh(core_axis_name="c", subcore_axis_name="s")

    @functools.partial(
        pl.kernel,
        mesh=mesh,
        out_type=jax.ShapeDtypeStruct((_NC * _NSEG, _D), jnp.float32),
        scratch_types=(
            [pltpu.VMEM_SHARED((_NSEG, _D), jnp.float32)]
            + [pltpu.VMEM((_CHUNK, _D), jnp.float32)] * _NBUF
            + [pltpu.VMEM((_NJ + 1, _CHUNK), jnp.int32),
               pltpu.VMEM((_TAIL, _D), jnp.float32),
               pltpu.VMEM((_TAIL,), jnp.int32)]
            + [pltpu.SemaphoreType.DMA] * (2 * _NBUF + 1)
        ),
    )
    def k(feat_hbm, ids1_hbm, out_hbm, acc, *rest):
        bufs = rest[:_NBUF]
        idx_v, tbuf, tidx = rest[_NBUF:_NBUF + 3]
        lsems = rest[_NBUF + 3:2 * _NBUF + 3]
        ssems = rest[2 * _NBUF + 3:3 * _NBUF + 3]
        isem = rest[3 * _NBUF + 3]
        cid = lax.axis_index("c")
        sid = lax.axis_index("s")
        wid = cid * _NS + sid
        has_extra = (wid % 5 == 0) | (wid % 5 == 2)
        rank = 2 * (wid // 5) + jnp.where(wid % 5 == 2, 1, 0)

        id_loads = []
        for j in range(_NJ + 1):
            row = ((_NJ * _NW + rank) if j == _NJ else (j * _NW + wid)) * _CHUNK
            id_loads.append(pltpu.async_copy(
                ids1_hbm.at[pl.ds(pl.multiple_of(row, _CHUNK), _CHUNK)],
                idx_v.at[j], isem))

        zero = jnp.zeros((16,), jnp.float32)
        for r in range(_ZROWS):
            for l in range(_D // 16):
                bufs[0][r, pl.ds(l * 16, 16)] = zero
        pltpu.sync_copy(bufs[0].at[pl.ds(0, _ZROWS)],
                        acc.at[pl.ds(sid * _ZROWS, _ZROWS)])

        def start_load(j):
            row = pl.multiple_of((j * _NW + wid) * _CHUNK, _CHUNK)
            return pltpu.async_copy(feat_hbm.at[pl.ds(row, _CHUNK)],
                                    bufs[j % _NBUF], lsems[j % _NBUF])

        loads = {j: start_load(j) for j in range(_AHEAD)}
        for h in id_loads:
            h.wait()
        plsc.subcore_barrier()
        scats = {}
        for j in range(_NJ):
            if j >= _AHEAD:
                scats.pop(j - _AHEAD).wait()
            if j + _AHEAD < _NJ:
                loads[j + _AHEAD] = start_load(j + _AHEAD)
            loads.pop(j).wait()
            scats[j] = pltpu.async_copy(bufs[j % _NBUF], acc.at[idx_v.at[j]],
                                        ssems[j % _NBUF], add=True)
        for j in sorted(scats):
            scats.pop(j).wait()

        @pl.when(has_extra)
        def _():
            row = pl.multiple_of((_NJ * _NW + rank) * _CHUNK, _CHUNK)
            pltpu.async_copy(feat_hbm.at[pl.ds(row, _CHUNK)],
                             bufs[_NJ % _NBUF], lsems[_NJ % _NBUF]).wait()
            pltpu.sync_copy(bufs[_NJ % _NBUF], acc.at[idx_v.at[_NJ]], add=True)

        @pl.when(wid == 1)
        def _():
            base = _NFULL * _CHUNK
            f = pltpu.async_copy(feat_hbm.at[pl.ds(base, _TAIL)], tbuf,
                                 lsems[0])
            i = pltpu.async_copy(ids1_hbm.at[pl.ds(base, _TAIL)], tidx,
                                 lsems[1])
            f.wait()
            i.wait()
            pltpu.sync_copy(tbuf, acc.at[tidx], add=True)

        plsc.subcore_barrier()
        pltpu.sync_copy(acc.at[pl.ds(sid * _ZROWS, _ZROWS)],
                        out_hbm.at[pl.ds(cid * _NSEG + sid * _ZROWS, _ZROWS)])

    return k(feat, ids_flat)


def _tc_merge(x_ref, o_ref):
    o_ref[...] = x_ref[0] + x_ref[1]


def kernel(feat, segment_ids, num_segments):
    ids = segment_ids.astype(jnp.int32)
    part = _sc_partial_segment_sum(feat, ids).reshape(_NC, _NSEG, _D)
    return pl.pallas_call(
        _tc_merge,
        out_shape=jax.ShapeDtypeStruct((_NSEG, _D), jnp.float32),
    )(part)
